# Initial kernel scaffold; baseline (speedup 1.0000x reference)
import functools

import jax
import jax.numpy as jnp
from jax import lax
from jax.experimental import pallas as pl
from jax.experimental.pallas import tpu as pltpu
from jax.experimental.pallas import tpu_sc as plsc

N_NODES = 10000
N_EDGES = 320000
D = 128
CPAD = 16
CHUNK = 128
NCHUNKS = N_EDGES // CHUNK

_SC_INFO = plsc.get_sparse_core_info()
NC = _SC_INFO.num_cores
NS = _SC_INFO.num_subcores
NW = NC * NS

ROWS_PER_TILE = N_NODES // NS
ZCH = 125


def _leaky(x):
    return jnp.where(x > 0, x, 0.2 * x)


def _sc_gather_body(p_hbm, q_hbm, c_hbm, row_hbm, col_hbm,
                    pr_hbm, qc_hbm, cr_hbm, cc_hbm,
                    idx_r, idx_c, buf_p, buf_q, buf_cr, buf_cc, sem):
    wid = lax.axis_index("s") * NC + lax.axis_index("c")
    n_iter = pl.cdiv(NCHUNKS, NW)

    def body(i, carry):
        cid = i * NW + wid

        @pl.when(cid < NCHUNKS)
        def _():
            base = cid * CHUNK
            pltpu.sync_copy(row_hbm.at[pl.ds(base, CHUNK)], idx_r)
            pltpu.sync_copy(col_hbm.at[pl.ds(base, CHUNK)], idx_c)
            cp1 = pltpu.async_copy(p_hbm.at[idx_r], buf_p, sem)
            cp1.wait()
            cp2 = pltpu.async_copy(q_hbm.at[idx_c], buf_q, sem)
            cp2.wait()
            cp3 = pltpu.async_copy(c_hbm.at[idx_r], buf_cr, sem)
            cp3.wait()
            cp4 = pltpu.async_copy(c_hbm.at[idx_c], buf_cc, sem)
            cp4.wait()
            pltpu.sync_copy(buf_p, pr_hbm.at[pl.ds(base, CHUNK)])
            pltpu.sync_copy(buf_q, qc_hbm.at[pl.ds(base, CHUNK)])
            pltpu.sync_copy(buf_cr, cr_hbm.at[pl.ds(base, CHUNK)])
            pltpu.sync_copy(buf_cc, cc_hbm.at[pl.ds(base, CHUNK)])

        return carry

    lax.fori_loop(0, n_iter, body, 0)


_sc_gather = functools.partial(
    pl.kernel,
    out_type=(
        jax.ShapeDtypeStruct((N_EDGES, D), jnp.float32),
        jax.ShapeDtypeStruct((N_EDGES, D), jnp.float32),
        jax.ShapeDtypeStruct((N_EDGES, CPAD), jnp.float32),
        jax.ShapeDtypeStruct((N_EDGES, CPAD), jnp.float32),
    ),
    mesh=plsc.VectorSubcoreMesh(core_axis_name="c", subcore_axis_name="s"),
    scratch_types=[
        pltpu.VMEM((CHUNK,), jnp.int32),
        pltpu.VMEM((CHUNK,), jnp.int32),
        pltpu.VMEM((CHUNK, D), jnp.float32),
        pltpu.VMEM((CHUNK, D), jnp.float32),
        pltpu.VMEM((CHUNK, CPAD), jnp.float32),
        pltpu.VMEM((CHUNK, CPAD), jnp.float32),
        pltpu.SemaphoreType.DMA,
    ],
)(_sc_gather_body)


def _sc_scatter_body(ef_hbm, row_hbm, zero_hbm, out_hbm,
                     idx_v, ef_v, z_v, agg_sh, sem):
    c = lax.axis_index("c")
    s = lax.axis_index("s")

    pltpu.sync_copy(zero_hbm, z_v)
    for j in range(ROWS_PER_TILE // ZCH):
        pltpu.sync_copy(z_v, agg_sh.at[pl.ds(s * ROWS_PER_TILE + j * ZCH, ZCH)])
    plsc.subcore_barrier()

    half = NCHUNKS // NC
    n_iter = pl.cdiv(half, NS)

    def body(i, carry):
        j = i * NS + s

        @pl.when(j < half)
        def _():
            cid = j * NC + c
            base = cid * CHUNK
            pltpu.sync_copy(row_hbm.at[pl.ds(base, CHUNK)], idx_v)
            pltpu.sync_copy(ef_hbm.at[pl.ds(base, CHUNK)], ef_v)
            pltpu.sync_copy(ef_v, agg_sh.at[idx_v], add=True)

        return carry

    lax.fori_loop(0, n_iter, body, 0)
    plsc.subcore_barrier()

    for j in range(ROWS_PER_TILE // ZCH):
        off = s * ROWS_PER_TILE + j * ZCH
        pltpu.sync_copy(agg_sh.at[pl.ds(off, ZCH)], ef_v.at[pl.ds(0, ZCH)])
        pltpu.sync_copy(ef_v.at[pl.ds(0, ZCH)], out_hbm.at[c, pl.ds(off, ZCH)])


_sc_scatter = functools.partial(
    pl.kernel,
    out_type=jax.ShapeDtypeStruct((NC, N_NODES, D), jnp.float32),
    mesh=plsc.VectorSubcoreMesh(core_axis_name="c", subcore_axis_name="s"),
    scratch_types=[
        pltpu.VMEM((CHUNK,), jnp.int32),
        pltpu.VMEM((CHUNK, D), jnp.float32),
        pltpu.VMEM((ZCH, D), jnp.float32),
        pltpu.VMEM_SHARED((N_NODES, D), jnp.float32),
        pltpu.SemaphoreType.DMA,
    ],
)(_sc_scatter_body)


def _tc_prep_body(h_ref, at_ref, bt_ref, p_ref, q_ref):
    hh = h_ref[...]
    p_ref[...] = jnp.dot(hh, at_ref[...], preferred_element_type=jnp.float32)
    q_ref[...] = jnp.dot(hh, bt_ref[...], preferred_element_type=jnp.float32)


def _tc_prep(h, At, Bt):
    return pl.pallas_call(
        _tc_prep_body,
        out_shape=(
            jax.ShapeDtypeStruct((N_NODES, D), jnp.float32),
            jax.ShapeDtypeStruct((N_NODES, D), jnp.float32),
        ),
    )(h, At, Bt)


BE = 4000


def _tc_edge_body(pr_ref, qc_ref, cr_ref, cc_ref, wr_ref, b1_ref, w2_ref,
                  b2_ref, out_ref):
    dd = cr_ref[...] - cc_ref[...]
    radial = jnp.sum(dd * dd, axis=1, keepdims=True)
    x = pr_ref[...] + qc_ref[...] + radial * wr_ref[...] + b1_ref[...]
    x = _leaky(x)
    y = jnp.dot(x, w2_ref[...], preferred_element_type=jnp.float32) + b2_ref[...]
    out_ref[...] = _leaky(y)


def _tc_edge(pr, qc, cr, cc, wr, be1, W2t, be2):
    grid = (N_EDGES // BE,)
    return pl.pallas_call(
        _tc_edge_body,
        grid=grid,
        in_specs=[
            pl.BlockSpec((BE, D), lambda i: (i, 0)),
            pl.BlockSpec((BE, D), lambda i: (i, 0)),
            pl.BlockSpec((BE, CPAD), lambda i: (i, 0)),
            pl.BlockSpec((BE, CPAD), lambda i: (i, 0)),
            pl.BlockSpec((1, D), lambda i: (0, 0)),
            pl.BlockSpec((1, D), lambda i: (0, 0)),
            pl.BlockSpec((D, D), lambda i: (0, 0)),
            pl.BlockSpec((1, D), lambda i: (0, 0)),
        ],
        out_specs=pl.BlockSpec((BE, D), lambda i: (i, 0)),
        out_shape=jax.ShapeDtypeStruct((N_EDGES, D), jnp.float32),
    )(pr, qc, cr, cc, wr, be1, W2t, be2)


def _tc_node_body(h_ref, agg_ref, w1h_ref, w1a_ref, b1_ref, w2_ref, b2_ref,
                  out_ref):
    hh = h_ref[...]
    agg = agg_ref[0] + agg_ref[1]
    x = (jnp.dot(hh, w1h_ref[...], preferred_element_type=jnp.float32)
         + jnp.dot(agg, w1a_ref[...], preferred_element_type=jnp.float32)
         + b1_ref[...])
    x = _leaky(x)
    y = jnp.dot(x, w2_ref[...], preferred_element_type=jnp.float32) + b2_ref[...]
    out_ref[...] = hh + y


def _tc_node(h, aggp, W1ht, W1at, bn1, W2t, bn2):
    return pl.pallas_call(
        _tc_node_body,
        out_shape=jax.ShapeDtypeStruct((N_NODES, D), jnp.float32),
    )(h, aggp, W1ht, W1at, bn1, W2t, bn2)


def kernel(h, edge_index, coord, We1, be1, We2, be2, Wn1, bn1, Wn2, bn2):
    f32 = jnp.float32
    row = edge_index[0].astype(jnp.int32)
    col = edge_index[1].astype(jnp.int32)
    coordpad = jnp.concatenate(
        [coord.astype(f32), jnp.zeros((N_NODES, CPAD - 3), f32)], axis=1)

    At = We1[:, :D].T
    Bt = We1[:, D:2 * D].T
    wr = We1[:, 2 * D].reshape(1, D)
    b1e = be1.reshape(1, D)
    W2t = We2.T
    b2e = be2.reshape(1, D)
    W1ht = Wn1[:, :D].T
    W1at = Wn1[:, D:].T
    b1n = bn1.reshape(1, D)
    W2nt = Wn2.T
    b2n = bn2.reshape(1, D)

    P, Q = _tc_prep(h, At, Bt)
    pr, qc, cr, cc = _sc_gather(P, Q, coordpad, row, col)
    edge_feat = _tc_edge(pr, qc, cr, cc, wr, b1e, W2t, b2e)
    zeros_tile = jnp.zeros((ZCH, D), f32)
    aggp = _sc_scatter(edge_feat, row, zeros_tile)
    h_out = _tc_node(h, aggp, W1ht, W1at, b1n, W2nt, b2n)
    return (h_out, coord, edge_feat)


# speedup vs baseline: 4.3159x; 4.3159x over previous
import functools

import jax
import jax.numpy as jnp
from jax import lax
from jax.experimental import pallas as pl
from jax.experimental.pallas import tpu as pltpu
from jax.experimental.pallas import tpu_sc as plsc

N_NODES = 10000
N_EDGES = 320000
D = 128
CPAD = 4
CHUNK = 128
NCHUNKS = N_EDGES // CHUNK
NGRP = CHUNK // 16

NC = 2
NS = 16
NW = NC * NS

ZCH = 200
NZCH = N_NODES // ZCH


def _leaky(x):
    return jnp.where(x > 0, x, 0.2 * x)


def _sc_gather_body(p_hbm, q_hbm, c_hbm, row_hbm, col_hbm,
                    pr_hbm, qc_hbm, rad_hbm,
                    idx_r, idx_c, buf_p, buf_q, coord_tab, buf_rad, sem):
    wid = lax.axis_index("s") * NC + lax.axis_index("c")
    n_iter = pl.cdiv(NCHUNKS, NW)

    pltpu.sync_copy(c_hbm, coord_tab)

    def body(i, carry):
        cid = i * NW + wid

        @pl.when(cid < NCHUNKS)
        def _():
            base = cid * CHUNK
            pltpu.sync_copy(row_hbm.at[pl.ds(base, CHUNK)], idx_r)
            pltpu.sync_copy(col_hbm.at[pl.ds(base, CHUNK)], idx_c)
            cp1 = pltpu.async_copy(p_hbm.at[idx_r], buf_p, sem)
            cp2 = pltpu.async_copy(q_hbm.at[idx_c], buf_q, sem)
            for g in range(NGRP):
                iv = idx_r[pl.ds(g * 16, 16)] * CPAD
                jv = idx_c[pl.ds(g * 16, 16)] * CPAD
                rad = None
                for k in range(3):
                    dk = (plsc.load_gather(coord_tab, [iv + k])
                          - plsc.load_gather(coord_tab, [jv + k]))
                    sq = dk * dk
                    rad = sq if rad is None else rad + sq
                buf_rad[pl.ds(g * 16, 16)] = rad
            cp1.wait()
            cp2.wait()
            pltpu.sync_copy(buf_p, pr_hbm.at[pl.ds(base, CHUNK)])
            pltpu.sync_copy(buf_q, qc_hbm.at[pl.ds(base, CHUNK)])
            pltpu.sync_copy(buf_rad, rad_hbm.at[pl.ds(base, CHUNK)])

        return carry

    lax.fori_loop(0, n_iter, body, 0)


@functools.lru_cache(maxsize=None)
def _sc_gather():
    return pl.kernel(
        _sc_gather_body,
        out_type=(
            jax.ShapeDtypeStruct((N_EDGES, D), jnp.float32),
            jax.ShapeDtypeStruct((N_EDGES, D), jnp.float32),
            jax.ShapeDtypeStruct((N_EDGES,), jnp.float32),
        ),
        mesh=plsc.VectorSubcoreMesh(
            core_axis_name="c", subcore_axis_name="s",
            num_cores=NC, num_subcores=NS),
        scratch_types=[
            pltpu.VMEM((CHUNK,), jnp.int32),
            pltpu.VMEM((CHUNK,), jnp.int32),
            pltpu.VMEM((CHUNK, D), jnp.float32),
            pltpu.VMEM((CHUNK, D), jnp.float32),
            pltpu.VMEM((N_NODES * CPAD,), jnp.float32),
            pltpu.VMEM((CHUNK,), jnp.float32),
            pltpu.SemaphoreType.DMA,
        ],
        compiler_params=pltpu.CompilerParams(needs_layout_passes=False),
    )


def _sc_scatter_body(ef_hbm, row_hbm, zero_hbm, out_hbm,
                     idx_v, ef_v, z_v, agg_sh, sem):
    c = lax.axis_index("c")
    s = lax.axis_index("s")

    pltpu.sync_copy(zero_hbm, z_v)
    for i in range(pl.cdiv(NZCH, NS)):
        j = i * NS + s

        @pl.when(j < NZCH)
        def _():
            pltpu.sync_copy(z_v, agg_sh.at[pl.ds(j * ZCH, ZCH)])

    plsc.subcore_barrier()

    half = NCHUNKS // NC
    n_iter = pl.cdiv(half, NS)

    def body(i, carry):
        j = i * NS + s

        @pl.when(j < half)
        def _():
            cid = j * NC + c
            base = cid * CHUNK
            pltpu.sync_copy(row_hbm.at[pl.ds(base, CHUNK)], idx_v)
            pltpu.sync_copy(ef_hbm.at[pl.ds(base, CHUNK)], ef_v)
            pltpu.sync_copy(ef_v, agg_sh.at[idx_v], add=True)

        return carry

    lax.fori_loop(0, n_iter, body, 0)
    plsc.subcore_barrier()

    for i in range(pl.cdiv(NZCH, NS)):
        j = i * NS + s

        @pl.when(j < NZCH)
        def _():
            off = j * ZCH
            pltpu.sync_copy(agg_sh.at[pl.ds(off, ZCH)], z_v)
            pltpu.sync_copy(z_v, out_hbm.at[c, pl.ds(off, ZCH)])


@functools.lru_cache(maxsize=None)
def _sc_scatter():
    return pl.kernel(
        _sc_scatter_body,
        out_type=jax.ShapeDtypeStruct((NC, N_NODES, D), jnp.float32),
        mesh=plsc.VectorSubcoreMesh(
            core_axis_name="c", subcore_axis_name="s",
            num_cores=NC, num_subcores=NS),
        scratch_types=[
            pltpu.VMEM((CHUNK,), jnp.int32),
            pltpu.VMEM((CHUNK, D), jnp.float32),
            pltpu.VMEM((ZCH, D), jnp.float32),
            pltpu.VMEM_SHARED((N_NODES, D), jnp.float32),
            pltpu.SemaphoreType.DMA,
        ],
    )


def _tc_prep_body(h_ref, at_ref, bt_ref, p_ref, q_ref):
    hh = h_ref[...]
    p_ref[...] = jnp.dot(hh, at_ref[...], preferred_element_type=jnp.float32)
    q_ref[...] = jnp.dot(hh, bt_ref[...], preferred_element_type=jnp.float32)


def _tc_prep(h, At, Bt):
    return pl.pallas_call(
        _tc_prep_body,
        out_shape=(
            jax.ShapeDtypeStruct((N_NODES, D), jnp.float32),
            jax.ShapeDtypeStruct((N_NODES, D), jnp.float32),
        ),
    )(h, At, Bt)


BE = 4000


def _tc_edge_body(pr_ref, qc_ref, rad_ref, wr_ref, b1_ref, w2_ref,
                  b2_ref, out_ref):
    radial = rad_ref[...]
    x = pr_ref[...] + qc_ref[...] + radial * wr_ref[...] + b1_ref[...]
    x = _leaky(x)
    y = jnp.dot(x, w2_ref[...], preferred_element_type=jnp.float32) + b2_ref[...]
    out_ref[...] = _leaky(y)


def _tc_edge(pr, qc, rad2d, wr, be1, W2t, be2):
    grid = (N_EDGES // BE,)
    return pl.pallas_call(
        _tc_edge_body,
        grid=grid,
        in_specs=[
            pl.BlockSpec((BE, D), lambda i: (i, 0)),
            pl.BlockSpec((BE, D), lambda i: (i, 0)),
            pl.BlockSpec((BE, 1), lambda i: (i, 0)),
            pl.BlockSpec((1, D), lambda i: (0, 0)),
            pl.BlockSpec((1, D), lambda i: (0, 0)),
            pl.BlockSpec((D, D), lambda i: (0, 0)),
            pl.BlockSpec((1, D), lambda i: (0, 0)),
        ],
        out_specs=pl.BlockSpec((BE, D), lambda i: (i, 0)),
        out_shape=jax.ShapeDtypeStruct((N_EDGES, D), jnp.float32),
    )(pr, qc, rad2d, wr, be1, W2t, be2)


def _tc_node_body(h_ref, agg_ref, w1h_ref, w1a_ref, b1_ref, w2_ref, b2_ref,
                  out_ref):
    hh = h_ref[...]
    agg = agg_ref[0] + agg_ref[1]
    x = (jnp.dot(hh, w1h_ref[...], preferred_element_type=jnp.float32)
         + jnp.dot(agg, w1a_ref[...], preferred_element_type=jnp.float32)
         + b1_ref[...])
    x = _leaky(x)
    y = jnp.dot(x, w2_ref[...], preferred_element_type=jnp.float32) + b2_ref[...]
    out_ref[...] = hh + y


def _tc_node(h, aggp, W1ht, W1at, bn1, W2t, bn2):
    return pl.pallas_call(
        _tc_node_body,
        out_shape=jax.ShapeDtypeStruct((N_NODES, D), jnp.float32),
    )(h, aggp, W1ht, W1at, bn1, W2t, bn2)


def kernel(h, edge_index, coord, We1, be1, We2, be2, Wn1, bn1, Wn2, bn2):
    f32 = jnp.float32
    row = edge_index[0].astype(jnp.int32)
    col = edge_index[1].astype(jnp.int32)
    coordpad = jnp.concatenate(
        [coord.astype(f32), jnp.zeros((N_NODES, CPAD - 3), f32)],
        axis=1).reshape(N_NODES * CPAD)

    At = We1[:, :D].T
    Bt = We1[:, D:2 * D].T
    wr = We1[:, 2 * D].reshape(1, D)
    b1e = be1.reshape(1, D)
    W2t = We2.T
    b2e = be2.reshape(1, D)
    W1ht = Wn1[:, :D].T
    W1at = Wn1[:, D:].T
    b1n = bn1.reshape(1, D)
    W2nt = Wn2.T
    b2n = bn2.reshape(1, D)

    P, Q = _tc_prep(h, At, Bt)
    pr, qc, rad = _sc_gather()(P, Q, coordpad, row, col)
    edge_feat = _tc_edge(pr, qc, rad.reshape(N_EDGES, 1), wr, b1e, W2t, b2e)
    zeros_tile = jnp.zeros((ZCH, D), f32)
    aggp = _sc_scatter()(edge_feat, row, zeros_tile)
    h_out = _tc_node(h, aggp, W1ht, W1at, b1n, W2nt, b2n)
    return (h_out, coord, edge_feat)


# speedup vs baseline: 5.5466x; 1.2852x over previous
import functools

import jax
import jax.numpy as jnp
from jax import lax
from jax.experimental import pallas as pl
from jax.experimental.pallas import tpu as pltpu
from jax.experimental.pallas import tpu_sc as plsc

N_NODES = 10000
N_EDGES = 320000
D = 128
CPAD = 4
CHUNK = 128
NCHUNKS = N_EDGES // CHUNK
NGRP = CHUNK // 16

NC = 2
NS = 16
NW = NC * NS

ZCH = 200
NZCH = N_NODES // ZCH


def _leaky(x):
    return jnp.where(x > 0, x, 0.2 * x)


NLOOP = 2 * pl.cdiv(pl.cdiv(NCHUNKS, NW), 2)


def _sc_gather_body(p_hbm, q_hbm, c_hbm, row_hbm, col_hbm, wr_hbm,
                    esum_hbm,
                    idx_r, idx_c, buf_p, buf_q, coord_tab, wr_tab,
                    gsem0, gsem1, wsem0, wsem1):
    wid = lax.axis_index("s") * NC + lax.axis_index("c")
    gsems = (gsem0, gsem1)
    wsems = (wsem0, wsem1)

    pltpu.sync_copy(c_hbm, coord_tab)
    pltpu.sync_copy(wr_hbm, wr_tab)
    wr_vecs = [wr_tab[pl.ds(k * 16, 16)] for k in range(D // 16)]

    def cid_of(i):
        return i * NW + wid

    def issue(b, cid):
        base = cid * CHUNK
        pltpu.sync_copy(row_hbm.at[pl.ds(base, CHUNK)], idx_r.at[b])
        pltpu.sync_copy(col_hbm.at[pl.ds(base, CHUNK)], idx_c.at[b])
        pltpu.async_copy(p_hbm.at[idx_r.at[b]], buf_p.at[b], gsems[b])
        pltpu.async_copy(q_hbm.at[idx_c.at[b]], buf_q.at[b], gsems[b])

    def wait_gathers(b):
        pltpu.make_async_copy(p_hbm.at[idx_r.at[b]], buf_p.at[b],
                              gsems[b]).wait()
        pltpu.make_async_copy(q_hbm.at[idx_c.at[b]], buf_q.at[b],
                              gsems[b]).wait()

    def write(b, cid):
        pltpu.async_copy(buf_p.at[b], esum_hbm.at[pl.ds(cid * CHUNK, CHUNK)],
                         wsems[b])

    def wait_write(b):
        pltpu.make_async_copy(buf_p.at[b], esum_hbm.at[pl.ds(0, CHUNK)],
                              wsems[b]).wait()

    def fuse(b):
        def grp_body(g, carry):
            iv = idx_r[b, pl.ds(g * 16, 16)] * CPAD
            jv = idx_c[b, pl.ds(g * 16, 16)] * CPAD
            rad = None
            for k in range(3):
                dk = (plsc.load_gather(coord_tab, [iv + k])
                      - plsc.load_gather(coord_tab, [jv + k]))
                sq = dk * dk
                rad = sq if rad is None else rad + sq
            for l in range(16):
                e = g * 16 + l
                r = rad[l]
                for k in range(D // 16):
                    sl = pl.ds(k * 16, 16)
                    buf_p[b, e, sl] = (buf_p[b, e, sl] + buf_q[b, e, sl]
                                       + r * wr_vecs[k])
            return carry

        lax.fori_loop(0, NGRP, grp_body, 0)

    issue(0, cid_of(0))

    def outer(j, carry):
        for b in (0, 1):
            i = j * 2 + b
            cur, nxt = b, 1 - b

            @pl.when(cid_of(i + 1) < NCHUNKS)
            def _():
                @pl.when(i >= 1)
                def _():
                    wait_write(nxt)
                issue(nxt, cid_of(i + 1))

            @pl.when(cid_of(i) < NCHUNKS)
            def _():
                wait_gathers(cur)
                fuse(cur)
                write(cur, cid_of(i))

        return carry

    lax.fori_loop(0, NLOOP // 2, outer, 0)
    wait_write(0)
    wait_write(1)


@functools.lru_cache(maxsize=None)
def _sc_gather():
    return pl.kernel(
        _sc_gather_body,
        out_type=jax.ShapeDtypeStruct((N_EDGES, D), jnp.float32),
        mesh=plsc.VectorSubcoreMesh(
            core_axis_name="c", subcore_axis_name="s",
            num_cores=NC, num_subcores=NS),
        scratch_types=[
            pltpu.VMEM((2, CHUNK), jnp.int32),
            pltpu.VMEM((2, CHUNK), jnp.int32),
            pltpu.VMEM((2, CHUNK, D), jnp.float32),
            pltpu.VMEM((2, CHUNK, D), jnp.float32),
            pltpu.VMEM((N_NODES * CPAD,), jnp.float32),
            pltpu.VMEM((D,), jnp.float32),
            pltpu.SemaphoreType.DMA,
            pltpu.SemaphoreType.DMA,
            pltpu.SemaphoreType.DMA,
            pltpu.SemaphoreType.DMA,
        ],
        compiler_params=pltpu.CompilerParams(needs_layout_passes=False),
    )


def _sc_scatter_body(ef_hbm, row_hbm, zero_hbm, out_hbm,
                     idx_v, ef_v, z_v, agg_sh, sem):
    c = lax.axis_index("c")
    s = lax.axis_index("s")

    pltpu.sync_copy(zero_hbm, z_v)
    for i in range(pl.cdiv(NZCH, NS)):
        j = i * NS + s

        @pl.when(j < NZCH)
        def _():
            pltpu.sync_copy(z_v, agg_sh.at[pl.ds(j * ZCH, ZCH)])

    plsc.subcore_barrier()

    half = NCHUNKS // NC
    n_iter = pl.cdiv(half, NS)

    def body(i, carry):
        j = i * NS + s

        @pl.when(j < half)
        def _():
            cid = j * NC + c
            base = cid * CHUNK
            pltpu.sync_copy(row_hbm.at[pl.ds(base, CHUNK)], idx_v)
            pltpu.sync_copy(ef_hbm.at[pl.ds(base, CHUNK)], ef_v)
            pltpu.sync_copy(ef_v, agg_sh.at[idx_v], add=True)

        return carry

    lax.fori_loop(0, n_iter, body, 0)
    plsc.subcore_barrier()

    for i in range(pl.cdiv(NZCH, NS)):
        j = i * NS + s

        @pl.when(j < NZCH)
        def _():
            off = j * ZCH
            pltpu.sync_copy(agg_sh.at[pl.ds(off, ZCH)], z_v)
            pltpu.sync_copy(z_v, out_hbm.at[c, pl.ds(off, ZCH)])


@functools.lru_cache(maxsize=None)
def _sc_scatter():
    return pl.kernel(
        _sc_scatter_body,
        out_type=jax.ShapeDtypeStruct((NC, N_NODES, D), jnp.float32),
        mesh=plsc.VectorSubcoreMesh(
            core_axis_name="c", subcore_axis_name="s",
            num_cores=NC, num_subcores=NS),
        scratch_types=[
            pltpu.VMEM((CHUNK,), jnp.int32),
            pltpu.VMEM((CHUNK, D), jnp.float32),
            pltpu.VMEM((ZCH, D), jnp.float32),
            pltpu.VMEM_SHARED((N_NODES, D), jnp.float32),
            pltpu.SemaphoreType.DMA,
        ],
    )


def _tc_prep_body(h_ref, at_ref, bt_ref, p_ref, q_ref):
    hh = h_ref[...]
    p_ref[...] = jnp.dot(hh, at_ref[...], preferred_element_type=jnp.float32)
    q_ref[...] = jnp.dot(hh, bt_ref[...], preferred_element_type=jnp.float32)


def _tc_prep(h, At, Bt):
    return pl.pallas_call(
        _tc_prep_body,
        out_shape=(
            jax.ShapeDtypeStruct((N_NODES, D), jnp.float32),
            jax.ShapeDtypeStruct((N_NODES, D), jnp.float32),
        ),
    )(h, At, Bt)


BE = 4000


def _tc_edge_body(es_ref, b1_ref, w2_ref, b2_ref, out_ref):
    x = _leaky(es_ref[...] + b1_ref[...])
    y = jnp.dot(x, w2_ref[...], preferred_element_type=jnp.float32) + b2_ref[...]
    out_ref[...] = _leaky(y)


def _tc_edge(esum, be1, W2t, be2):
    grid = (N_EDGES // BE,)
    return pl.pallas_call(
        _tc_edge_body,
        grid=grid,
        in_specs=[
            pl.BlockSpec((BE, D), lambda i: (i, 0)),
            pl.BlockSpec((1, D), lambda i: (0, 0)),
            pl.BlockSpec((D, D), lambda i: (0, 0)),
            pl.BlockSpec((1, D), lambda i: (0, 0)),
        ],
        out_specs=pl.BlockSpec((BE, D), lambda i: (i, 0)),
        out_shape=jax.ShapeDtypeStruct((N_EDGES, D), jnp.float32),
    )(esum, be1, W2t, be2)


def _tc_node_body(h_ref, agg_ref, w1h_ref, w1a_ref, b1_ref, w2_ref, b2_ref,
                  out_ref):
    hh = h_ref[...]
    agg = agg_ref[0] + agg_ref[1]
    x = (jnp.dot(hh, w1h_ref[...], preferred_element_type=jnp.float32)
         + jnp.dot(agg, w1a_ref[...], preferred_element_type=jnp.float32)
         + b1_ref[...])
    x = _leaky(x)
    y = jnp.dot(x, w2_ref[...], preferred_element_type=jnp.float32) + b2_ref[...]
    out_ref[...] = hh + y


def _tc_node(h, aggp, W1ht, W1at, bn1, W2t, bn2):
    return pl.pallas_call(
        _tc_node_body,
        out_shape=jax.ShapeDtypeStruct((N_NODES, D), jnp.float32),
    )(h, aggp, W1ht, W1at, bn1, W2t, bn2)


def kernel(h, edge_index, coord, We1, be1, We2, be2, Wn1, bn1, Wn2, bn2):
    f32 = jnp.float32
    row = edge_index[0].astype(jnp.int32)
    col = edge_index[1].astype(jnp.int32)
    coordpad = jnp.concatenate(
        [coord.astype(f32), jnp.zeros((N_NODES, CPAD - 3), f32)],
        axis=1).reshape(N_NODES * CPAD)

    At = We1[:, :D].T
    Bt = We1[:, D:2 * D].T
    wr = We1[:, 2 * D].reshape(1, D)
    b1e = be1.reshape(1, D)
    W2t = We2.T
    b2e = be2.reshape(1, D)
    W1ht = Wn1[:, :D].T
    W1at = Wn1[:, D:].T
    b1n = bn1.reshape(1, D)
    W2nt = Wn2.T
    b2n = bn2.reshape(1, D)

    P, Q = _tc_prep(h, At, Bt)
    esum = _sc_gather()(P, Q, coordpad, row, col, We1[:, 2 * D])
    edge_feat = _tc_edge(esum, b1e, W2t, b2e)
    zeros_tile = jnp.zeros((ZCH, D), f32)
    aggp = _sc_scatter()(edge_feat, row, zeros_tile)
    h_out = _tc_node(h, aggp, W1ht, W1at, b1n, W2nt, b2n)
    return (h_out, coord, edge_feat)


# speedup vs baseline: 5.6025x; 1.0101x over previous
import functools

import jax
import jax.numpy as jnp
from jax import lax
from jax.experimental import pallas as pl
from jax.experimental.pallas import tpu as pltpu
from jax.experimental.pallas import tpu_sc as plsc

N_NODES = 10000
N_EDGES = 320000
D = 128
CPAD = 4
CHUNK = 128
NCHUNKS = N_EDGES // CHUNK
NGRP = CHUNK // 16

NC = 2
NS = 16
NW = NC * NS

ZCH = 200
NZCH = N_NODES // ZCH


def _leaky(x):
    return jnp.where(x > 0, x, 0.2 * x)


UNROLL = 6
NLOOP = UNROLL * pl.cdiv(pl.cdiv(NCHUNKS, NW), UNROLL)


def _sc_gather_body(p_hbm, q_hbm, cx_hbm, cy_hbm, cz_hbm, rc_hbm,
                    esum_hbm, rad_hbm,
                    idx_v, buf_p, buf_q, buf_rad, cx_t, cy_t, cz_t,
                    gsem0, gsem1, wsem0, wsem1, wsem2):
    wid = lax.axis_index("s") * NC + lax.axis_index("c")
    gsems = (gsem0, gsem1)
    wsems = (wsem0, wsem1, wsem2)

    pltpu.sync_copy(cx_hbm, cx_t)
    pltpu.sync_copy(cy_hbm, cy_t)
    pltpu.sync_copy(cz_hbm, cz_t)

    def cid_of(i):
        return i * NW + wid

    def issue(qs, ps, cid):
        pltpu.sync_copy(rc_hbm.at[cid], idx_v.at[qs])
        pltpu.async_copy(p_hbm.at[idx_v.at[qs, 0]], buf_p.at[ps], gsems[qs])
        pltpu.async_copy(q_hbm.at[idx_v.at[qs, 1]], buf_q.at[qs], gsems[qs])

    def wait_gathers(qs, ps):
        pltpu.make_async_copy(p_hbm.at[idx_v.at[qs, 0]], buf_p.at[ps],
                              gsems[qs]).wait()
        pltpu.make_async_copy(q_hbm.at[idx_v.at[qs, 1]], buf_q.at[qs],
                              gsems[qs]).wait()

    def write(ps, cid):
        pltpu.async_copy(buf_p.at[ps], esum_hbm.at[pl.ds(cid * CHUNK, CHUNK)],
                         wsems[ps])
        pltpu.async_copy(buf_rad.at[ps], rad_hbm.at[pl.ds(cid * CHUNK, CHUNK)],
                         wsems[ps])

    def wait_write(ps):
        pltpu.make_async_copy(buf_p.at[ps], esum_hbm.at[pl.ds(0, CHUNK)],
                              wsems[ps]).wait()
        pltpu.make_async_copy(buf_rad.at[ps], rad_hbm.at[pl.ds(0, CHUNK)],
                              wsems[ps]).wait()

    def fuse(qs, ps):
        def grp_body(g, carry):
            iv = idx_v[qs, 0, pl.ds(g * 16, 16)]
            jv = idx_v[qs, 1, pl.ds(g * 16, 16)]
            dx = plsc.load_gather(cx_t, [iv]) - plsc.load_gather(cx_t, [jv])
            dy = plsc.load_gather(cy_t, [iv]) - plsc.load_gather(cy_t, [jv])
            dz = plsc.load_gather(cz_t, [iv]) - plsc.load_gather(cz_t, [jv])
            buf_rad[ps, pl.ds(g * 16, 16)] = dx * dx + dy * dy + dz * dz
            return carry

        lax.fori_loop(0, NGRP, grp_body, 0)

        def add_body(e, carry):
            for k in range(D // 16):
                sl = pl.ds(k * 16, 16)
                plsc.addupdate(buf_p.at[ps, e, sl], buf_q[qs, e, sl])
            return carry

        lax.fori_loop(0, CHUNK, add_body, 0)

    issue(0, 0, cid_of(0))

    def outer(j, carry):
        for t in range(UNROLL):
            i = j * UNROLL + t
            qs, ps = t % 2, t % 3
            qs_n, ps_n = (t + 1) % 2, (t + 1) % 3

            @pl.when(cid_of(i + 1) < NCHUNKS)
            def _():
                @pl.when(i >= 2)
                def _():
                    wait_write(ps_n)
                issue(qs_n, ps_n, cid_of(i + 1))

            @pl.when(cid_of(i) < NCHUNKS)
            def _():
                wait_gathers(qs, ps)
                fuse(qs, ps)
                write(ps, cid_of(i))

        return carry

    lax.fori_loop(0, NLOOP // UNROLL, outer, 0)
    wait_write(0)
    wait_write(1)
    wait_write(2)


@functools.lru_cache(maxsize=None)
def _sc_gather():
    return pl.kernel(
        _sc_gather_body,
        out_type=(
            jax.ShapeDtypeStruct((N_EDGES, D), jnp.float32),
            jax.ShapeDtypeStruct((N_EDGES,), jnp.float32),
        ),
        mesh=plsc.VectorSubcoreMesh(
            core_axis_name="c", subcore_axis_name="s",
            num_cores=NC, num_subcores=NS),
        scratch_types=[
            pltpu.VMEM((2, 2, CHUNK), jnp.int32),
            pltpu.VMEM((3, CHUNK, D), jnp.float32),
            pltpu.VMEM((2, CHUNK, D), jnp.float32),
            pltpu.VMEM((3, CHUNK), jnp.float32),
            pltpu.VMEM((N_NODES,), jnp.float32),
            pltpu.VMEM((N_NODES,), jnp.float32),
            pltpu.VMEM((N_NODES,), jnp.float32),
            pltpu.SemaphoreType.DMA,
            pltpu.SemaphoreType.DMA,
            pltpu.SemaphoreType.DMA,
            pltpu.SemaphoreType.DMA,
            pltpu.SemaphoreType.DMA,
        ],
        compiler_params=pltpu.CompilerParams(needs_layout_passes=False),
    )


def _sc_scatter_body(ef_hbm, row_hbm, zero_hbm, out_hbm,
                     idx_v, ef_v, z_v, agg_sh, sem):
    c = lax.axis_index("c")
    s = lax.axis_index("s")

    pltpu.sync_copy(zero_hbm, z_v)
    for i in range(pl.cdiv(NZCH, NS)):
        j = i * NS + s

        @pl.when(j < NZCH)
        def _():
            pltpu.sync_copy(z_v, agg_sh.at[pl.ds(j * ZCH, ZCH)])

    plsc.subcore_barrier()

    half = NCHUNKS // NC
    n_iter = pl.cdiv(half, NS)

    def body(i, carry):
        j = i * NS + s

        @pl.when(j < half)
        def _():
            cid = j * NC + c
            base = cid * CHUNK
            pltpu.sync_copy(row_hbm.at[pl.ds(base, CHUNK)], idx_v)
            pltpu.sync_copy(ef_hbm.at[pl.ds(base, CHUNK)], ef_v)
            pltpu.sync_copy(ef_v, agg_sh.at[idx_v], add=True)

        return carry

    lax.fori_loop(0, n_iter, body, 0)
    plsc.subcore_barrier()

    for i in range(pl.cdiv(NZCH, NS)):
        j = i * NS + s

        @pl.when(j < NZCH)
        def _():
            off = j * ZCH
            pltpu.sync_copy(agg_sh.at[pl.ds(off, ZCH)], z_v)
            pltpu.sync_copy(z_v, out_hbm.at[c, pl.ds(off, ZCH)])


@functools.lru_cache(maxsize=None)
def _sc_scatter():
    return pl.kernel(
        _sc_scatter_body,
        out_type=jax.ShapeDtypeStruct((NC, N_NODES, D), jnp.float32),
        mesh=plsc.VectorSubcoreMesh(
            core_axis_name="c", subcore_axis_name="s",
            num_cores=NC, num_subcores=NS),
        scratch_types=[
            pltpu.VMEM((CHUNK,), jnp.int32),
            pltpu.VMEM((CHUNK, D), jnp.float32),
            pltpu.VMEM((ZCH, D), jnp.float32),
            pltpu.VMEM_SHARED((N_NODES, D), jnp.float32),
            pltpu.SemaphoreType.DMA,
        ],
    )


def _tc_prep_body(h_ref, at_ref, bt_ref, p_ref, q_ref):
    hh = h_ref[...]
    p_ref[...] = jnp.dot(hh, at_ref[...], preferred_element_type=jnp.float32)
    q_ref[...] = jnp.dot(hh, bt_ref[...], preferred_element_type=jnp.float32)


def _tc_prep(h, At, Bt):
    return pl.pallas_call(
        _tc_prep_body,
        out_shape=(
            jax.ShapeDtypeStruct((N_NODES, D), jnp.float32),
            jax.ShapeDtypeStruct((N_NODES, D), jnp.float32),
        ),
    )(h, At, Bt)


BE = 4000


def _tc_edge_body(es_ref, rad_ref, wr_ref, b1_ref, w2_ref, b2_ref, out_ref):
    x = _leaky(es_ref[...] + rad_ref[...] * wr_ref[...] + b1_ref[...])
    y = jnp.dot(x, w2_ref[...], preferred_element_type=jnp.float32) + b2_ref[...]
    out_ref[...] = _leaky(y)


def _tc_edge(esum, rad2d, wr, be1, W2t, be2):
    grid = (N_EDGES // BE,)
    return pl.pallas_call(
        _tc_edge_body,
        grid=grid,
        in_specs=[
            pl.BlockSpec((BE, D), lambda i: (i, 0)),
            pl.BlockSpec((BE, 1), lambda i: (i, 0)),
            pl.BlockSpec((1, D), lambda i: (0, 0)),
            pl.BlockSpec((1, D), lambda i: (0, 0)),
            pl.BlockSpec((D, D), lambda i: (0, 0)),
            pl.BlockSpec((1, D), lambda i: (0, 0)),
        ],
        out_specs=pl.BlockSpec((BE, D), lambda i: (i, 0)),
        out_shape=jax.ShapeDtypeStruct((N_EDGES, D), jnp.float32),
    )(esum, rad2d, wr, be1, W2t, be2)


def _tc_node_body(h_ref, agg_ref, w1h_ref, w1a_ref, b1_ref, w2_ref, b2_ref,
                  out_ref):
    hh = h_ref[...]
    agg = agg_ref[0] + agg_ref[1]
    x = (jnp.dot(hh, w1h_ref[...], preferred_element_type=jnp.float32)
         + jnp.dot(agg, w1a_ref[...], preferred_element_type=jnp.float32)
         + b1_ref[...])
    x = _leaky(x)
    y = jnp.dot(x, w2_ref[...], preferred_element_type=jnp.float32) + b2_ref[...]
    out_ref[...] = hh + y


def _tc_node(h, aggp, W1ht, W1at, bn1, W2t, bn2):
    return pl.pallas_call(
        _tc_node_body,
        out_shape=jax.ShapeDtypeStruct((N_NODES, D), jnp.float32),
    )(h, aggp, W1ht, W1at, bn1, W2t, bn2)


def kernel(h, edge_index, coord, We1, be1, We2, be2, Wn1, bn1, Wn2, bn2):
    f32 = jnp.float32
    row = edge_index[0].astype(jnp.int32)
    col = edge_index[1].astype(jnp.int32)
    rc = jnp.stack([row.reshape(NCHUNKS, CHUNK),
                    col.reshape(NCHUNKS, CHUNK)], axis=1)
    cx = coord[:, 0].astype(f32)
    cy = coord[:, 1].astype(f32)
    cz = coord[:, 2].astype(f32)

    At = We1[:, :D].T
    Bt = We1[:, D:2 * D].T
    wr = We1[:, 2 * D].reshape(1, D)
    b1e = be1.reshape(1, D)
    W2t = We2.T
    b2e = be2.reshape(1, D)
    W1ht = Wn1[:, :D].T
    W1at = Wn1[:, D:].T
    b1n = bn1.reshape(1, D)
    W2nt = Wn2.T
    b2n = bn2.reshape(1, D)

    P, Q = _tc_prep(h, At, Bt)
    esum, rad = _sc_gather()(P, Q, cx, cy, cz, rc)
    edge_feat = _tc_edge(esum, rad.reshape(N_EDGES, 1), wr, b1e, W2t, b2e)
    zeros_tile = jnp.zeros((ZCH, D), f32)
    aggp = _sc_scatter()(edge_feat, row, zeros_tile)
    h_out = _tc_node(h, aggp, W1ht, W1at, b1n, W2nt, b2n)
    return (h_out, coord, edge_feat)


# speedup vs baseline: 6.1935x; 1.1055x over previous
import functools

import jax
import jax.numpy as jnp
from jax import lax
from jax.experimental import pallas as pl
from jax.experimental.pallas import tpu as pltpu
from jax.experimental.pallas import tpu_sc as plsc

N_NODES = 10000
N_EDGES = 320000
D = 128
CPAD = 4
CHUNK = 128
NCHUNKS = N_EDGES // CHUNK
NGRP = CHUNK // 16

NC = 2
NS = 16
NW = NC * NS

ZCH = 200
NZCH = N_NODES // ZCH


def _leaky(x):
    return jnp.where(x > 0, x, 0.2 * x)


UNROLL = 6
NLOOP = UNROLL * pl.cdiv(pl.cdiv(NCHUNKS, NW), UNROLL)


def _sc_gather_body(p_hbm, q_hbm, cx_hbm, cy_hbm, cz_hbm, rc_hbm, wr_hbm,
                    esum_hbm,
                    idx_v, buf_p, buf_q, cx_t, cy_t, cz_t, wr_t,
                    gsem0, gsem1, wsem0, wsem1, wsem2):
    wid = lax.axis_index("s") * NC + lax.axis_index("c")
    gsems = (gsem0, gsem1)
    wsems = (wsem0, wsem1, wsem2)

    pltpu.sync_copy(cx_hbm, cx_t)
    pltpu.sync_copy(cy_hbm, cy_t)
    pltpu.sync_copy(cz_hbm, cz_t)
    pltpu.sync_copy(wr_hbm, wr_t)
    wr_vecs = [wr_t[pl.ds(k * 16, 16)] for k in range(D // 16)]

    def cid_of(i):
        return i * NW + wid

    def issue(qs, ps, cid):
        pltpu.sync_copy(rc_hbm.at[cid], idx_v.at[qs])
        pltpu.async_copy(p_hbm.at[idx_v.at[qs, 0]], buf_p.at[ps], gsems[qs])
        pltpu.async_copy(q_hbm.at[idx_v.at[qs, 1]], buf_q.at[qs], gsems[qs])

    def wait_gathers(qs, ps):
        pltpu.make_async_copy(p_hbm.at[idx_v.at[qs, 0]], buf_p.at[ps],
                              gsems[qs]).wait()
        pltpu.make_async_copy(q_hbm.at[idx_v.at[qs, 1]], buf_q.at[qs],
                              gsems[qs]).wait()

    def write(ps, cid):
        pltpu.async_copy(buf_p.at[ps], esum_hbm.at[pl.ds(cid * CHUNK, CHUNK)],
                         wsems[ps])

    def wait_write(ps):
        pltpu.make_async_copy(buf_p.at[ps], esum_hbm.at[pl.ds(0, CHUNK)],
                              wsems[ps]).wait()

    def fuse(qs, ps):
        def grp_body(g, carry):
            iv = idx_v[qs, 0, pl.ds(g * 16, 16)]
            jv = idx_v[qs, 1, pl.ds(g * 16, 16)]
            dx = plsc.load_gather(cx_t, [iv]) - plsc.load_gather(cx_t, [jv])
            dy = plsc.load_gather(cy_t, [iv]) - plsc.load_gather(cy_t, [jv])
            dz = plsc.load_gather(cz_t, [iv]) - plsc.load_gather(cz_t, [jv])
            rad = dx * dx + dy * dy + dz * dz
            for l in range(16):
                e = g * 16 + l
                r = rad[l]
                for k in range(D // 16):
                    sl = pl.ds(k * 16, 16)
                    plsc.addupdate(buf_p.at[ps, e, sl],
                                   buf_q[qs, e, sl] + r * wr_vecs[k])
            return carry

        lax.fori_loop(0, NGRP, grp_body, 0)

    issue(0, 0, cid_of(0))

    def outer(j, carry):
        for t in range(UNROLL):
            i = j * UNROLL + t
            qs, ps = t % 2, t % 3
            qs_n, ps_n = (t + 1) % 2, (t + 1) % 3

            @pl.when(cid_of(i + 1) < NCHUNKS)
            def _():
                @pl.when(i >= 2)
                def _():
                    wait_write(ps_n)
                issue(qs_n, ps_n, cid_of(i + 1))

            @pl.when(cid_of(i) < NCHUNKS)
            def _():
                wait_gathers(qs, ps)
                fuse(qs, ps)
                write(ps, cid_of(i))

        return carry

    lax.fori_loop(0, NLOOP // UNROLL, outer, 0)
    wait_write(0)
    wait_write(1)
    wait_write(2)


@functools.lru_cache(maxsize=None)
def _sc_gather():
    return pl.kernel(
        _sc_gather_body,
        out_type=jax.ShapeDtypeStruct((N_EDGES, D), jnp.float32),
        mesh=plsc.VectorSubcoreMesh(
            core_axis_name="c", subcore_axis_name="s",
            num_cores=NC, num_subcores=NS),
        scratch_types=[
            pltpu.VMEM((2, 2, CHUNK), jnp.int32),
            pltpu.VMEM((3, CHUNK, D), jnp.float32),
            pltpu.VMEM((2, CHUNK, D), jnp.float32),
            pltpu.VMEM((N_NODES,), jnp.float32),
            pltpu.VMEM((N_NODES,), jnp.float32),
            pltpu.VMEM((N_NODES,), jnp.float32),
            pltpu.VMEM((D,), jnp.float32),
            pltpu.SemaphoreType.DMA,
            pltpu.SemaphoreType.DMA,
            pltpu.SemaphoreType.DMA,
            pltpu.SemaphoreType.DMA,
            pltpu.SemaphoreType.DMA,
        ],
        compiler_params=pltpu.CompilerParams(needs_layout_passes=False),
    )


def _sc_scatter_body(ef_hbm, row_hbm, zero_hbm, out_hbm,
                     idx_v, ef_v, z_v, agg_sh, sem):
    c = lax.axis_index("c")
    s = lax.axis_index("s")

    pltpu.sync_copy(zero_hbm, z_v)
    for i in range(pl.cdiv(NZCH, NS)):
        j = i * NS + s

        @pl.when(j < NZCH)
        def _():
            pltpu.sync_copy(z_v, agg_sh.at[pl.ds(j * ZCH, ZCH)])

    plsc.subcore_barrier()

    half = NCHUNKS // NC
    n_iter = pl.cdiv(half, NS)

    def body(i, carry):
        j = i * NS + s

        @pl.when(j < half)
        def _():
            cid = j * NC + c
            base = cid * CHUNK
            pltpu.sync_copy(row_hbm.at[pl.ds(base, CHUNK)], idx_v)
            pltpu.sync_copy(ef_hbm.at[pl.ds(base, CHUNK)], ef_v)
            pltpu.sync_copy(ef_v, agg_sh.at[idx_v], add=True)

        return carry

    lax.fori_loop(0, n_iter, body, 0)
    plsc.subcore_barrier()

    for i in range(pl.cdiv(NZCH, NS)):
        j = i * NS + s

        @pl.when(j < NZCH)
        def _():
            off = j * ZCH
            pltpu.sync_copy(agg_sh.at[pl.ds(off, ZCH)], z_v)
            pltpu.sync_copy(z_v, out_hbm.at[c, pl.ds(off, ZCH)])


@functools.lru_cache(maxsize=None)
def _sc_scatter():
    return pl.kernel(
        _sc_scatter_body,
        out_type=jax.ShapeDtypeStruct((NC, N_NODES, D), jnp.float32),
        mesh=plsc.VectorSubcoreMesh(
            core_axis_name="c", subcore_axis_name="s",
            num_cores=NC, num_subcores=NS),
        scratch_types=[
            pltpu.VMEM((CHUNK,), jnp.int32),
            pltpu.VMEM((CHUNK, D), jnp.float32),
            pltpu.VMEM((ZCH, D), jnp.float32),
            pltpu.VMEM_SHARED((N_NODES, D), jnp.float32),
            pltpu.SemaphoreType.DMA,
        ],
    )


def _tc_prep_body(h_ref, at_ref, bt_ref, p_ref, q_ref):
    hh = h_ref[...]
    p_ref[...] = jnp.dot(hh, at_ref[...], preferred_element_type=jnp.float32)
    q_ref[...] = jnp.dot(hh, bt_ref[...], preferred_element_type=jnp.float32)


def _tc_prep(h, At, Bt):
    return pl.pallas_call(
        _tc_prep_body,
        out_shape=(
            jax.ShapeDtypeStruct((N_NODES, D), jnp.float32),
            jax.ShapeDtypeStruct((N_NODES, D), jnp.float32),
        ),
    )(h, At, Bt)


BE = 4000


def _tc_edge_body(es_ref, b1_ref, w2_ref, b2_ref, out_ref):
    x = _leaky(es_ref[...] + b1_ref[...])
    y = jnp.dot(x, w2_ref[...], preferred_element_type=jnp.float32) + b2_ref[...]
    out_ref[...] = _leaky(y)


def _tc_edge(esum, be1, W2t, be2):
    grid = (N_EDGES // BE,)
    return pl.pallas_call(
        _tc_edge_body,
        grid=grid,
        in_specs=[
            pl.BlockSpec((BE, D), lambda i: (i, 0)),
            pl.BlockSpec((1, D), lambda i: (0, 0)),
            pl.BlockSpec((D, D), lambda i: (0, 0)),
            pl.BlockSpec((1, D), lambda i: (0, 0)),
        ],
        out_specs=pl.BlockSpec((BE, D), lambda i: (i, 0)),
        out_shape=jax.ShapeDtypeStruct((N_EDGES, D), jnp.float32),
    )(esum, be1, W2t, be2)


def _tc_node_body(h_ref, agg_ref, w1h_ref, w1a_ref, b1_ref, w2_ref, b2_ref,
                  out_ref):
    hh = h_ref[...]
    agg = agg_ref[0] + agg_ref[1]
    x = (jnp.dot(hh, w1h_ref[...], preferred_element_type=jnp.float32)
         + jnp.dot(agg, w1a_ref[...], preferred_element_type=jnp.float32)
         + b1_ref[...])
    x = _leaky(x)
    y = jnp.dot(x, w2_ref[...], preferred_element_type=jnp.float32) + b2_ref[...]
    out_ref[...] = hh + y


def _tc_node(h, aggp, W1ht, W1at, bn1, W2t, bn2):
    return pl.pallas_call(
        _tc_node_body,
        out_shape=jax.ShapeDtypeStruct((N_NODES, D), jnp.float32),
    )(h, aggp, W1ht, W1at, bn1, W2t, bn2)


def kernel(h, edge_index, coord, We1, be1, We2, be2, Wn1, bn1, Wn2, bn2):
    f32 = jnp.float32
    row = edge_index[0].astype(jnp.int32)
    col = edge_index[1].astype(jnp.int32)
    rc = jnp.stack([row.reshape(NCHUNKS, CHUNK),
                    col.reshape(NCHUNKS, CHUNK)], axis=1)
    cx = coord[:, 0].astype(f32)
    cy = coord[:, 1].astype(f32)
    cz = coord[:, 2].astype(f32)

    At = We1[:, :D].T
    Bt = We1[:, D:2 * D].T
    wr = We1[:, 2 * D].reshape(1, D)
    b1e = be1.reshape(1, D)
    W2t = We2.T
    b2e = be2.reshape(1, D)
    W1ht = Wn1[:, :D].T
    W1at = Wn1[:, D:].T
    b1n = bn1.reshape(1, D)
    W2nt = Wn2.T
    b2n = bn2.reshape(1, D)

    P, Q = _tc_prep(h, At, Bt)
    esum = _sc_gather()(P, Q, cx, cy, cz, rc, wr.reshape(D))
    edge_feat = _tc_edge(esum, b1e, W2t, b2e)
    zeros_tile = jnp.zeros((ZCH, D), f32)
    aggp = _sc_scatter()(edge_feat, row, zeros_tile)
    h_out = _tc_node(h, aggp, W1ht, W1at, b1n, W2nt, b2n)
    return (h_out, coord, edge_feat)


# speedup vs baseline: 7.5556x; 1.2199x over previous
import functools

import jax
import jax.numpy as jnp
from jax import lax
from jax.experimental import pallas as pl
from jax.experimental.pallas import tpu as pltpu
from jax.experimental.pallas import tpu_sc as plsc

N_NODES = 10000
N_EDGES = 320000
D = 128
CPAD = 4
CHUNK = 128
NCHUNKS = N_EDGES // CHUNK
NGRP = CHUNK // 16

NC = 2
NS = 16
NW = NC * NS

ZCH = 80
NZCH = N_NODES // ZCH


def _leaky(x):
    return jnp.where(x > 0, x, 0.2 * x)


UNROLL = 6
NLOOP = UNROLL * pl.cdiv(pl.cdiv(NCHUNKS, NW), UNROLL)


def _sc_gather_body(p_hbm, q_hbm, cx_hbm, cy_hbm, cz_hbm, rc_hbm, wr_hbm,
                    esum_hbm,
                    idx_v, buf_p, buf_q, cx_t, cy_t, cz_t, wr_t,
                    gsem0, gsem1, wsem0, wsem1, wsem2):
    wid = lax.axis_index("s") * NC + lax.axis_index("c")
    gsems = (gsem0, gsem1)
    wsems = (wsem0, wsem1, wsem2)

    pltpu.sync_copy(cx_hbm, cx_t)
    pltpu.sync_copy(cy_hbm, cy_t)
    pltpu.sync_copy(cz_hbm, cz_t)
    pltpu.sync_copy(wr_hbm, wr_t)
    wr_vecs = [wr_t[pl.ds(k * 16, 16)] for k in range(D // 16)]

    def cid_of(i):
        return i * NW + wid

    def issue(qs, ps, cid):
        pltpu.sync_copy(rc_hbm.at[cid], idx_v.at[qs])
        pltpu.async_copy(p_hbm.at[idx_v.at[qs, 0]], buf_p.at[ps], gsems[qs])
        pltpu.async_copy(q_hbm.at[idx_v.at[qs, 1]], buf_q.at[qs], gsems[qs])

    def wait_gathers(qs, ps):
        pltpu.make_async_copy(p_hbm.at[idx_v.at[qs, 0]], buf_p.at[ps],
                              gsems[qs]).wait()
        pltpu.make_async_copy(q_hbm.at[idx_v.at[qs, 1]], buf_q.at[qs],
                              gsems[qs]).wait()

    def write(ps, cid):
        pltpu.async_copy(buf_p.at[ps], esum_hbm.at[pl.ds(cid * CHUNK, CHUNK)],
                         wsems[ps])

    def wait_write(ps):
        pltpu.make_async_copy(buf_p.at[ps], esum_hbm.at[pl.ds(0, CHUNK)],
                              wsems[ps]).wait()

    def fuse(qs, ps):
        def grp_body(g, carry):
            iv = idx_v[qs, 0, pl.ds(g * 16, 16)]
            jv = idx_v[qs, 1, pl.ds(g * 16, 16)]
            dx = plsc.load_gather(cx_t, [iv]) - plsc.load_gather(cx_t, [jv])
            dy = plsc.load_gather(cy_t, [iv]) - plsc.load_gather(cy_t, [jv])
            dz = plsc.load_gather(cz_t, [iv]) - plsc.load_gather(cz_t, [jv])
            rad = dx * dx + dy * dy + dz * dz
            for l in range(16):
                e = g * 16 + l
                r = rad[l]
                for k in range(D // 16):
                    sl = pl.ds(k * 16, 16)
                    plsc.addupdate(buf_p.at[ps, e, sl],
                                   buf_q[qs, e, sl] + r * wr_vecs[k])
            return carry

        lax.fori_loop(0, NGRP, grp_body, 0)

    issue(0, 0, cid_of(0))

    def outer(j, carry):
        for t in range(UNROLL):
            i = j * UNROLL + t
            qs, ps = t % 2, t % 3
            qs_n, ps_n = (t + 1) % 2, (t + 1) % 3

            @pl.when(cid_of(i + 1) < NCHUNKS)
            def _():
                @pl.when(i >= 2)
                def _():
                    wait_write(ps_n)
                issue(qs_n, ps_n, cid_of(i + 1))

            @pl.when(cid_of(i) < NCHUNKS)
            def _():
                wait_gathers(qs, ps)
                fuse(qs, ps)
                write(ps, cid_of(i))

        return carry

    lax.fori_loop(0, NLOOP // UNROLL, outer, 0)
    wait_write(0)
    wait_write(1)
    wait_write(2)


@functools.lru_cache(maxsize=None)
def _sc_gather():
    return pl.kernel(
        _sc_gather_body,
        out_type=jax.ShapeDtypeStruct((N_EDGES, D), jnp.float32),
        mesh=plsc.VectorSubcoreMesh(
            core_axis_name="c", subcore_axis_name="s",
            num_cores=NC, num_subcores=NS),
        scratch_types=[
            pltpu.VMEM((2, 2, CHUNK), jnp.int32),
            pltpu.VMEM((3, CHUNK, D), jnp.float32),
            pltpu.VMEM((2, CHUNK, D), jnp.float32),
            pltpu.VMEM((N_NODES,), jnp.float32),
            pltpu.VMEM((N_NODES,), jnp.float32),
            pltpu.VMEM((N_NODES,), jnp.float32),
            pltpu.VMEM((D,), jnp.float32),
            pltpu.SemaphoreType.DMA,
            pltpu.SemaphoreType.DMA,
            pltpu.SemaphoreType.DMA,
            pltpu.SemaphoreType.DMA,
            pltpu.SemaphoreType.DMA,
        ],
        compiler_params=pltpu.CompilerParams(needs_layout_passes=False),
    )


SC_HALF = NCHUNKS // NC
NSTEPS = 2 * pl.cdiv(pl.cdiv(SC_HALF, NS), 2)


def _sc_scatter_body(ef_hbm, rc_hbm, zero_hbm, out_hbm,
                     idx_v, ef_v, agg_sh,
                     rsem0, rsem1, ssem0, ssem1):
    c = lax.axis_index("c")
    s = lax.axis_index("s")
    rsems = (rsem0, rsem1)
    ssems = (ssem0, ssem1)

    z_v = ef_v.at[0, pl.ds(0, ZCH)]
    pltpu.sync_copy(zero_hbm, z_v)
    for i in range(pl.cdiv(NZCH, NS)):
        j = i * NS + s

        @pl.when(j < NZCH)
        def _():
            pltpu.sync_copy(z_v, agg_sh.at[pl.ds(j * ZCH, ZCH)])

    plsc.subcore_barrier()

    def jm_of(i):
        return i * NS + s

    def issue_read(b, m):
        pltpu.async_copy(rc_hbm.at[m], idx_v.at[b], rsems[b])
        pltpu.async_copy(ef_hbm.at[pl.ds(m * CHUNK, CHUNK)], ef_v.at[b],
                         rsems[b])

    def wait_read(b):
        pltpu.make_async_copy(rc_hbm.at[0], idx_v.at[b], rsems[b]).wait()
        pltpu.make_async_copy(ef_hbm.at[pl.ds(0, CHUNK)], ef_v.at[b],
                              rsems[b]).wait()

    def issue_scatter(b):
        pltpu.async_copy(ef_v.at[b], agg_sh.at[idx_v.at[b, 0]], ssems[b],
                         add=True)

    def wait_scatter(b):
        pltpu.make_async_copy(ef_v.at[b], agg_sh.at[idx_v.at[b, 0]],
                              ssems[b]).wait()

    issue_read(0, jm_of(0) * NC + c)

    def outer(jj, carry):
        for t in range(2):
            i = jj * 2 + t
            b, nb = t, 1 - t

            @pl.when(jm_of(i + 1) < SC_HALF)
            def _():
                @pl.when(i >= 1)
                def _():
                    wait_scatter(nb)
                issue_read(nb, jm_of(i + 1) * NC + c)

            @pl.when(jm_of(i) < SC_HALF)
            def _():
                wait_read(b)
                issue_scatter(b)

        return carry

    lax.fori_loop(0, NSTEPS // 2, outer, 0)
    wait_scatter(0)
    wait_scatter(1)
    plsc.subcore_barrier()

    for i in range(pl.cdiv(NZCH, NS)):
        j = i * NS + s

        @pl.when(j < NZCH)
        def _():
            off = j * ZCH
            pltpu.sync_copy(agg_sh.at[pl.ds(off, ZCH)], z_v)
            pltpu.sync_copy(z_v, out_hbm.at[c, pl.ds(off, ZCH)])


@functools.lru_cache(maxsize=None)
def _sc_scatter():
    return pl.kernel(
        _sc_scatter_body,
        out_type=jax.ShapeDtypeStruct((NC, N_NODES, D), jnp.float32),
        mesh=plsc.VectorSubcoreMesh(
            core_axis_name="c", subcore_axis_name="s",
            num_cores=NC, num_subcores=NS),
        scratch_types=[
            pltpu.VMEM((2, 2, CHUNK), jnp.int32),
            pltpu.VMEM((2, CHUNK, D), jnp.float32),
            pltpu.VMEM_SHARED((N_NODES, D), jnp.float32),
            pltpu.SemaphoreType.DMA,
            pltpu.SemaphoreType.DMA,
            pltpu.SemaphoreType.DMA,
            pltpu.SemaphoreType.DMA,
        ],
    )


def _tc_prep_body(h_ref, at_ref, bt_ref, p_ref, q_ref):
    hh = h_ref[...]
    p_ref[...] = jnp.dot(hh, at_ref[...], preferred_element_type=jnp.float32)
    q_ref[...] = jnp.dot(hh, bt_ref[...], preferred_element_type=jnp.float32)


def _tc_prep(h, At, Bt):
    return pl.pallas_call(
        _tc_prep_body,
        out_shape=(
            jax.ShapeDtypeStruct((N_NODES, D), jnp.float32),
            jax.ShapeDtypeStruct((N_NODES, D), jnp.float32),
        ),
    )(h, At, Bt)


BE = 4000


def _tc_edge_body(es_ref, b1_ref, w2_ref, b2_ref, out_ref):
    x = _leaky(es_ref[...] + b1_ref[...])
    y = jnp.dot(x, w2_ref[...], preferred_element_type=jnp.float32) + b2_ref[...]
    out_ref[...] = _leaky(y)


def _tc_edge(esum, be1, W2t, be2):
    grid = (N_EDGES // BE,)
    return pl.pallas_call(
        _tc_edge_body,
        grid=grid,
        in_specs=[
            pl.BlockSpec((BE, D), lambda i: (i, 0)),
            pl.BlockSpec((1, D), lambda i: (0, 0)),
            pl.BlockSpec((D, D), lambda i: (0, 0)),
            pl.BlockSpec((1, D), lambda i: (0, 0)),
        ],
        out_specs=pl.BlockSpec((BE, D), lambda i: (i, 0)),
        out_shape=jax.ShapeDtypeStruct((N_EDGES, D), jnp.float32),
    )(esum, be1, W2t, be2)


def _tc_node_body(h_ref, agg_ref, w1h_ref, w1a_ref, b1_ref, w2_ref, b2_ref,
                  out_ref):
    hh = h_ref[...]
    agg = agg_ref[0] + agg_ref[1]
    x = (jnp.dot(hh, w1h_ref[...], preferred_element_type=jnp.float32)
         + jnp.dot(agg, w1a_ref[...], preferred_element_type=jnp.float32)
         + b1_ref[...])
    x = _leaky(x)
    y = jnp.dot(x, w2_ref[...], preferred_element_type=jnp.float32) + b2_ref[...]
    out_ref[...] = hh + y


def _tc_node(h, aggp, W1ht, W1at, bn1, W2t, bn2):
    return pl.pallas_call(
        _tc_node_body,
        out_shape=jax.ShapeDtypeStruct((N_NODES, D), jnp.float32),
    )(h, aggp, W1ht, W1at, bn1, W2t, bn2)


def kernel(h, edge_index, coord, We1, be1, We2, be2, Wn1, bn1, Wn2, bn2):
    f32 = jnp.float32
    row = edge_index[0].astype(jnp.int32)
    col = edge_index[1].astype(jnp.int32)
    rc = jnp.stack([row.reshape(NCHUNKS, CHUNK),
                    col.reshape(NCHUNKS, CHUNK)], axis=1)
    cx = coord[:, 0].astype(f32)
    cy = coord[:, 1].astype(f32)
    cz = coord[:, 2].astype(f32)

    At = We1[:, :D].T
    Bt = We1[:, D:2 * D].T
    wr = We1[:, 2 * D].reshape(1, D)
    b1e = be1.reshape(1, D)
    W2t = We2.T
    b2e = be2.reshape(1, D)
    W1ht = Wn1[:, :D].T
    W1at = Wn1[:, D:].T
    b1n = bn1.reshape(1, D)
    W2nt = Wn2.T
    b2n = bn2.reshape(1, D)

    P, Q = _tc_prep(h, At, Bt)
    esum = _sc_gather()(P, Q, cx, cy, cz, rc, wr.reshape(D))
    edge_feat = _tc_edge(esum, b1e, W2t, b2e)
    zeros_tile = jnp.zeros((ZCH, D), f32)
    aggp = _sc_scatter()(edge_feat, rc, zeros_tile)
    h_out = _tc_node(h, aggp, W1ht, W1at, b1n, W2nt, b2n)
    return (h_out, coord, edge_feat)
